# baseline (speedup 1.0000x reference)
import functools

import jax
import jax.numpy as jnp
from jax import lax
from jax.experimental import pallas as pl
from jax.experimental.pallas import tpu as pltpu
from jax.experimental.pallas import tpu_sc as plsc

BN_EPS = 1e-5

NC = 2
NS = 16
NW = NC * NS
CHUNK = 128


def _sc_mesh():
    return plsc.VectorSubcoreMesh(core_axis_name="c", subcore_axis_name="s")


def _make_deg_kernel(n, k, acc_rows):
    zpt = acc_rows // NS

    @functools.partial(
        pl.kernel,
        out_type=jax.ShapeDtypeStruct((NC, acc_rows, 16), jnp.float32),
        mesh=_sc_mesh(),
        compiler_params=pltpu.CompilerParams(use_tc_tiling_on_sc=False),
        scratch_types=[
            pltpu.VMEM((k, CHUNK), jnp.int32),
            pltpu.VMEM((CHUNK, 16), jnp.float32),
            pltpu.VMEM((CHUNK, 16), jnp.float32),
            pltpu.VMEM_SHARED((acc_rows, 16), jnp.float32),
        ],
    )
    def deg_kernel(dst_hbm, degp_hbm, dst_v, ones_v, zero_v, acc):
        cid = lax.axis_index("c")
        sid = lax.axis_index("s")
        wid = cid * NS + sid

        def _fill(r, carry):
            ones_v[r, :] = jnp.ones((16,), jnp.float32)
            zero_v[r, :] = jnp.zeros((16,), jnp.float32)
            return carry

        lax.fori_loop(0, CHUNK, _fill, 0)

        def _zero(c, carry):
            pltpu.sync_copy(zero_v, acc.at[pl.ds(sid * zpt + c * CHUNK, CHUNK)])
            return carry

        lax.fori_loop(0, zpt // CHUNK, _zero, 0)
        plsc.subcore_barrier()

        pltpu.sync_copy(dst_hbm.at[wid], dst_v)

        def _step(j, carry):
            pltpu.sync_copy(ones_v, acc.at[dst_v.at[j]], add=True)
            return carry

        lax.fori_loop(0, k, _step, 0)
        plsc.subcore_barrier()

        pltpu.sync_copy(acc.at[pl.ds(sid * zpt, zpt)],
                        degp_hbm.at[cid, pl.ds(sid * zpt, zpt)])

    return deg_kernel


def _make_edge_kernel(n, d, k, acc_rows):
    zpt = acc_rows // NS
    kh = -(-k // 2)
    kh += (-kh) % 8
    phases = [(0, min(kh, k)), (kh, k - kh)] if k > kh else [(0, k)]

    @functools.partial(
        pl.kernel,
        out_type=jax.ShapeDtypeStruct((NC, acc_rows, d), jnp.float32),
        mesh=_sc_mesh(),
        scratch_types=[
            pltpu.VMEM((kh, CHUNK), jnp.int32),
            pltpu.VMEM((kh, CHUNK), jnp.int32),
            pltpu.VMEM((CHUNK, d), jnp.float32),
            pltpu.VMEM((CHUNK, d), jnp.float32),
            pltpu.VMEM_SHARED((acc_rows, d), jnp.float32),
            pltpu.SemaphoreType.DMA,
            pltpu.SemaphoreType.DMA,
            pltpu.SemaphoreType.DMA,
            pltpu.SemaphoreType.DMA,
        ],
    )
    def edge_kernel(hp_hbm, src_hbm, dst_hbm, p_hbm,
                    src_v, dst_v, rows0, rows1, acc, g0, g1, s0, s1):
        cid = lax.axis_index("c")
        sid = lax.axis_index("s")
        wid = cid * NS + sid
        rows = (rows0, rows1)
        gsem = (g0, g1)
        ssem = (s0, s1)

        def _fill(r, carry):
            for c in range(d // 16):
                rows0[r, pl.ds(c * 16, 16)] = jnp.zeros((16,), jnp.float32)
            return carry

        lax.fori_loop(0, CHUNK, _fill, 0)

        def _zero(c, carry):
            pltpu.sync_copy(rows0, acc.at[pl.ds(sid * zpt + c * CHUNK, CHUNK)])
            return carry

        lax.fori_loop(0, zpt // CHUNK, _zero, 0)
        plsc.subcore_barrier()

        def _gather(j, b):
            pltpu.async_copy(hp_hbm.at[src_v.at[j]], rows[b], gsem[b])

        def _gather_wait(j, b):
            pltpu.make_async_copy(hp_hbm.at[src_v.at[j]], rows[b],
                                  gsem[b]).wait()

        for pi, (j0, cnt) in enumerate(phases):
            pltpu.sync_copy(src_hbm.at[wid, pl.ds(j0, cnt)],
                            src_v.at[pl.ds(0, cnt)])
            pltpu.sync_copy(dst_hbm.at[wid, pl.ds(j0, cnt)],
                            dst_v.at[pl.ds(0, cnt)])
            _gather(0, 0)

            def _step(j, carry):
                even = lax.rem(j, 2) == 0

                for b in range(2):
                    cond = even if b == 0 else jnp.logical_not(even)

                    @pl.when(cond)
                    def _body():
                        @pl.when(j + 1 < cnt)
                        def _pre():
                            _gather(j + 1, 1 - b)

                        _gather_wait(j, b)
                        pltpu.sync_copy(rows[b], acc.at[dst_v.at[j]],
                                        add=True)

                return carry

            lax.fori_loop(0, cnt, _step, 0)
        plsc.subcore_barrier()

        pltpu.sync_copy(acc.at[pl.ds(sid * zpt, zpt)],
                        p_hbm.at[cid, pl.ds(sid * zpt, zpt)])

    return edge_kernel


def _mm_body(x_ref, w_ref, dg_ref, hp_ref):
    deg = dg_ref[0, :, 0:1] + dg_ref[1, :, 0:1] + 1.0
    dinv = lax.rsqrt(deg)
    h = jnp.dot(x_ref[...], w_ref[...], preferred_element_type=jnp.float32)
    hp_ref[...] = h * dinv


def _matmul_scaled(x, w, degp, rows_blk=1000):
    n, d_in = x.shape
    d_out = w.shape[1]
    grid = n // rows_blk
    return pl.pallas_call(
        _mm_body,
        grid=(grid,),
        in_specs=[
            pl.BlockSpec((rows_blk, d_in), lambda i: (i, 0)),
            pl.BlockSpec((d_in, d_out), lambda i: (0, 0)),
            pl.BlockSpec((2, rows_blk, 16), lambda i: (0, i, 0)),
        ],
        out_specs=pl.BlockSpec((rows_blk, d_out), lambda i: (i, 0)),
        out_shape=jax.ShapeDtypeStruct((n, d_out), jnp.float32),
    )(x, w, degp)


def _post_body(p_ref, hp_ref, dg_ref, b_ref, g_ref, bt_ref, out_ref):
    n = hp_ref.shape[0]
    deg = dg_ref[0, 0:n, 0:1] + dg_ref[1, 0:n, 0:1] + 1.0
    dinv = lax.rsqrt(deg)
    o = (p_ref[0, 0:n, :] + p_ref[1, 0:n, :] + hp_ref[...]) * dinv + b_ref[...]
    o = jnp.maximum(o, 0.0)
    m = jnp.sum(o, axis=0, keepdims=True) * (1.0 / n)
    c = o - m
    v = jnp.sum(c * c, axis=0, keepdims=True) * (1.0 / n)
    out_ref[...] = c * lax.rsqrt(v + BN_EPS) * g_ref[...] + bt_ref[...]


def _postprocess(p, hp, degp, bias, gamma, beta):
    n, d = hp.shape
    return pl.pallas_call(
        _post_body,
        out_shape=jax.ShapeDtypeStruct((n, d), jnp.float32),
    )(p, hp, degp, bias.reshape(1, d), gamma.reshape(1, d), beta.reshape(1, d))


def kernel(x, edge_index, W, bias, gamma, beta):
    n, d_in = x.shape
    d = W.shape[1]
    e = edge_index.shape[1]

    k = -(-e // (NW * CHUNK))
    e_pad = NW * k * CHUNK
    pad = e_pad - e
    acc_rows = -(-(n + 1) // (NS * CHUNK)) * (NS * CHUNK)

    src = edge_index[0]
    dst = edge_index[1]
    if pad:
        fill = jnp.arange(pad, dtype=jnp.int32)
        src = jnp.concatenate([src, fill % n])
        dst = jnp.concatenate([dst, n + fill % (acc_rows - n)])
    src3 = src.reshape(NW, k, CHUNK)
    dst3 = dst.reshape(NW, k, CHUNK)

    degp = _make_deg_kernel(n, k, acc_rows)(dst3)
    hp = _matmul_scaled(x, W, degp)
    p = _make_edge_kernel(n, d, k, acc_rows)(hp, src3, dst3)
    return _postprocess(p, hp, degp, bias, gamma, beta)


# speedup vs baseline: 1.0208x; 1.0208x over previous
import functools

import jax
import jax.numpy as jnp
from jax import lax
from jax.experimental import pallas as pl
from jax.experimental.pallas import tpu as pltpu
from jax.experimental.pallas import tpu_sc as plsc

BN_EPS = 1e-5

NC = 2
NS = 16
NW = NC * NS
CHUNK = 128


def _sc_mesh():
    return plsc.VectorSubcoreMesh(core_axis_name="c", subcore_axis_name="s")


def _make_deg_kernel(n, k, acc_rows):
    zpt = acc_rows // NS

    @functools.partial(
        pl.kernel,
        out_type=jax.ShapeDtypeStruct((NC, acc_rows, 16), jnp.float32),
        mesh=_sc_mesh(),
        compiler_params=pltpu.CompilerParams(use_tc_tiling_on_sc=False),
        scratch_types=[
            pltpu.VMEM((k, CHUNK), jnp.int32),
            pltpu.VMEM((CHUNK, 16), jnp.float32),
            pltpu.VMEM((CHUNK, 16), jnp.float32),
            pltpu.VMEM_SHARED((acc_rows, 16), jnp.float32),
            pltpu.SemaphoreType.DMA,
        ],
    )
    def deg_kernel(dst_hbm, degp_hbm, dst_v, ones_v, zero_v, acc, sem):
        cid = lax.axis_index("c")
        sid = lax.axis_index("s")
        wid = cid * NS + sid

        def _fill(r, carry):
            ones_v[r, :] = jnp.ones((16,), jnp.float32)
            zero_v[r, :] = jnp.zeros((16,), jnp.float32)
            return carry

        lax.fori_loop(0, CHUNK, _fill, 0)

        def _zero(c, carry):
            pltpu.sync_copy(zero_v, acc.at[pl.ds(sid * zpt + c * CHUNK, CHUNK)])
            return carry

        lax.fori_loop(0, zpt // CHUNK, _zero, 0)
        plsc.subcore_barrier()

        pltpu.sync_copy(dst_hbm.at[wid], dst_v)

        def _fire(j, carry):
            pltpu.async_copy(ones_v, acc.at[dst_v.at[j]], sem, add=True)
            return carry

        lax.fori_loop(0, k, _fire, 0)

        def _drain(j, carry):
            pltpu.make_async_copy(ones_v, acc.at[dst_v.at[j]], sem).wait()
            return carry

        lax.fori_loop(0, k, _drain, 0)
        plsc.subcore_barrier()

        pltpu.sync_copy(acc.at[pl.ds(sid * zpt, zpt)],
                        degp_hbm.at[cid, pl.ds(sid * zpt, zpt)])

    return deg_kernel


def _make_edge_kernel(n, d, k, acc_rows):
    zpt = acc_rows // NS
    kh = -(-k // 2)
    kh += (-kh) % 8
    phases = [(0, min(kh, k)), (kh, k - kh)] if k > kh else [(0, k)]

    @functools.partial(
        pl.kernel,
        out_type=jax.ShapeDtypeStruct((NC, acc_rows, d), jnp.float32),
        mesh=_sc_mesh(),
        scratch_types=[
            pltpu.VMEM((kh, CHUNK), jnp.int32),
            pltpu.VMEM((kh, CHUNK), jnp.int32),
            pltpu.VMEM((CHUNK, d), jnp.float32),
            pltpu.VMEM((CHUNK, d), jnp.float32),
            pltpu.VMEM_SHARED((acc_rows, d), jnp.float32),
            pltpu.SemaphoreType.DMA,
            pltpu.SemaphoreType.DMA,
            pltpu.SemaphoreType.DMA,
            pltpu.SemaphoreType.DMA,
        ],
    )
    def edge_kernel(hp_hbm, src_hbm, dst_hbm, p_hbm,
                    src_v, dst_v, rows0, rows1, acc, g0, g1, s0, s1):
        cid = lax.axis_index("c")
        sid = lax.axis_index("s")
        wid = cid * NS + sid
        rows = (rows0, rows1)
        gsem = (g0, g1)
        ssem = (s0, s1)

        def _fill(r, carry):
            for c in range(d // 16):
                rows0[r, pl.ds(c * 16, 16)] = jnp.zeros((16,), jnp.float32)
            return carry

        lax.fori_loop(0, CHUNK, _fill, 0)

        def _zero(c, carry):
            pltpu.sync_copy(rows0, acc.at[pl.ds(sid * zpt + c * CHUNK, CHUNK)])
            return carry

        lax.fori_loop(0, zpt // CHUNK, _zero, 0)
        plsc.subcore_barrier()

        def _gather(j, b):
            pltpu.async_copy(hp_hbm.at[src_v.at[j]], rows[b], gsem[b])

        def _gather_wait(j, b):
            pltpu.make_async_copy(hp_hbm.at[src_v.at[j]], rows[b],
                                  gsem[b]).wait()

        for pi, (j0, cnt) in enumerate(phases):
            pltpu.sync_copy(src_hbm.at[wid, pl.ds(j0, cnt)],
                            src_v.at[pl.ds(0, cnt)])
            pltpu.sync_copy(dst_hbm.at[wid, pl.ds(j0, cnt)],
                            dst_v.at[pl.ds(0, cnt)])
            _gather(0, 0)

            def _step(j, carry):
                even = lax.rem(j, 2) == 0

                for b in range(2):
                    cond = even if b == 0 else jnp.logical_not(even)

                    @pl.when(cond)
                    def _body():
                        @pl.when(j + 1 < cnt)
                        def _pre():
                            _gather(j + 1, 1 - b)

                        _gather_wait(j, b)
                        pltpu.sync_copy(rows[b], acc.at[dst_v.at[j]],
                                        add=True)

                return carry

            lax.fori_loop(0, cnt, _step, 0)
        plsc.subcore_barrier()

        pltpu.sync_copy(acc.at[pl.ds(sid * zpt, zpt)],
                        p_hbm.at[cid, pl.ds(sid * zpt, zpt)])

    return edge_kernel


def _mm_body(x_ref, w_ref, dg_ref, hp_ref):
    deg = dg_ref[0, :, 0:1] + dg_ref[1, :, 0:1] + 1.0
    dinv = lax.rsqrt(deg)
    h = jnp.dot(x_ref[...], w_ref[...], preferred_element_type=jnp.float32)
    hp_ref[...] = h * dinv


def _matmul_scaled(x, w, degp, rows_blk=1000):
    n, d_in = x.shape
    d_out = w.shape[1]
    grid = n // rows_blk
    return pl.pallas_call(
        _mm_body,
        grid=(grid,),
        in_specs=[
            pl.BlockSpec((rows_blk, d_in), lambda i: (i, 0)),
            pl.BlockSpec((d_in, d_out), lambda i: (0, 0)),
            pl.BlockSpec((2, rows_blk, 16), lambda i: (0, i, 0)),
        ],
        out_specs=pl.BlockSpec((rows_blk, d_out), lambda i: (i, 0)),
        out_shape=jax.ShapeDtypeStruct((n, d_out), jnp.float32),
    )(x, w, degp)


def _post_body(p_ref, hp_ref, dg_ref, b_ref, g_ref, bt_ref, out_ref):
    n = hp_ref.shape[0]
    deg = dg_ref[0, 0:n, 0:1] + dg_ref[1, 0:n, 0:1] + 1.0
    dinv = lax.rsqrt(deg)
    o = (p_ref[0, 0:n, :] + p_ref[1, 0:n, :] + hp_ref[...]) * dinv + b_ref[...]
    o = jnp.maximum(o, 0.0)
    m = jnp.sum(o, axis=0, keepdims=True) * (1.0 / n)
    c = o - m
    v = jnp.sum(c * c, axis=0, keepdims=True) * (1.0 / n)
    out_ref[...] = c * lax.rsqrt(v + BN_EPS) * g_ref[...] + bt_ref[...]


def _postprocess(p, hp, degp, bias, gamma, beta):
    n, d = hp.shape
    return pl.pallas_call(
        _post_body,
        out_shape=jax.ShapeDtypeStruct((n, d), jnp.float32),
    )(p, hp, degp, bias.reshape(1, d), gamma.reshape(1, d), beta.reshape(1, d))


def kernel(x, edge_index, W, bias, gamma, beta):
    n, d_in = x.shape
    d = W.shape[1]
    e = edge_index.shape[1]

    k = -(-e // (NW * CHUNK))
    e_pad = NW * k * CHUNK
    pad = e_pad - e
    acc_rows = -(-(n + 1) // (NS * CHUNK)) * (NS * CHUNK)

    src = edge_index[0]
    dst = edge_index[1]
    if pad:
        fill = jnp.arange(pad, dtype=jnp.int32)
        src = jnp.concatenate([src, fill % n])
        dst = jnp.concatenate([dst, n + fill % (acc_rows - n)])
    src3 = src.reshape(NW, k, CHUNK)
    dst3 = dst.reshape(NW, k, CHUNK)

    degp = _make_deg_kernel(n, k, acc_rows)(dst3)
    hp = _matmul_scaled(x, W, degp)
    p = _make_edge_kernel(n, d, k, acc_rows)(hp, src3, dst3)
    return _postprocess(p, hp, degp, bias, gamma, beta)


# speedup vs baseline: 1.0503x; 1.0289x over previous
import functools

import jax
import jax.numpy as jnp
from jax import lax
from jax.experimental import pallas as pl
from jax.experimental.pallas import tpu as pltpu
from jax.experimental.pallas import tpu_sc as plsc

BN_EPS = 1e-5

NC = 2
NS = 16
NW = NC * NS
CHUNK = 128


def _sc_mesh():
    return plsc.VectorSubcoreMesh(core_axis_name="c", subcore_axis_name="s")


def _make_deg_kernel(n, k, acc_rows):
    zpt = acc_rows // NS

    @functools.partial(
        pl.kernel,
        out_type=jax.ShapeDtypeStruct((NC, acc_rows, 16), jnp.float32),
        mesh=_sc_mesh(),
        compiler_params=pltpu.CompilerParams(use_tc_tiling_on_sc=False),
        scratch_types=[
            pltpu.VMEM((k, CHUNK), jnp.int32),
            pltpu.VMEM((CHUNK, 16), jnp.float32),
            pltpu.VMEM((CHUNK, 16), jnp.float32),
            pltpu.VMEM_SHARED((acc_rows, 16), jnp.float32),
            pltpu.SemaphoreType.DMA,
        ],
    )
    def deg_kernel(dst_hbm, degp_hbm, dst_v, ones_v, zero_v, acc, sem):
        cid = lax.axis_index("c")
        sid = lax.axis_index("s")
        wid = cid * NS + sid

        def _fill(r, carry):
            ones_v[r, :] = jnp.ones((16,), jnp.float32)
            zero_v[r, :] = jnp.zeros((16,), jnp.float32)
            return carry

        lax.fori_loop(0, CHUNK, _fill, 0)

        def _zero(c, carry):
            pltpu.sync_copy(zero_v, acc.at[pl.ds(sid * zpt + c * CHUNK, CHUNK)])
            return carry

        lax.fori_loop(0, zpt // CHUNK, _zero, 0)
        plsc.subcore_barrier()

        pltpu.sync_copy(dst_hbm.at[wid], dst_v)

        def _fire(j, carry):
            pltpu.async_copy(ones_v, acc.at[dst_v.at[j]], sem, add=True)
            return carry

        lax.fori_loop(0, k, _fire, 0)

        def _drain(j, carry):
            pltpu.make_async_copy(ones_v, acc.at[dst_v.at[j]], sem).wait()
            return carry

        lax.fori_loop(0, k, _drain, 0)
        plsc.subcore_barrier()

        pltpu.sync_copy(acc.at[pl.ds(sid * zpt, zpt)],
                        degp_hbm.at[cid, pl.ds(sid * zpt, zpt)])

    return deg_kernel


ECHUNK = 112
EPH = 24


def _make_edge_kernel(n, d, k, acc_rows):
    zpt = acc_rows // NS
    phases = []
    j0 = 0
    while j0 < k:
        phases.append((j0, min(EPH, k - j0)))
        j0 += EPH
    assert all(cnt % 3 == 0 for _, cnt in phases)

    @functools.partial(
        pl.kernel,
        out_type=jax.ShapeDtypeStruct((NC, acc_rows, d), jnp.float32),
        mesh=_sc_mesh(),
        scratch_types=[
            pltpu.VMEM((EPH, ECHUNK), jnp.int32),
            pltpu.VMEM((EPH, ECHUNK), jnp.int32),
            [pltpu.VMEM((ECHUNK, d), jnp.float32) for _ in range(3)],
            pltpu.VMEM_SHARED((acc_rows, d), jnp.float32),
            [pltpu.SemaphoreType.DMA for _ in range(3)],
            [pltpu.SemaphoreType.DMA for _ in range(3)],
        ],
    )
    def edge_kernel(hp_hbm, src_hbm, dst_hbm, p_hbm,
                    src_v, dst_v, rows, acc, gsem, ssem):
        cid = lax.axis_index("c")
        sid = lax.axis_index("s")
        wid = cid * NS + sid

        def _fill(r, carry):
            for c in range(d // 16):
                rows[0][r, pl.ds(c * 16, 16)] = jnp.zeros((16,), jnp.float32)
            return carry

        lax.fori_loop(0, ECHUNK, _fill, 0)

        def _zero(c, carry):
            pltpu.sync_copy(rows[0],
                            acc.at[pl.ds(sid * zpt + c * ECHUNK, ECHUNK)])
            return carry

        lax.fori_loop(0, zpt // ECHUNK, _zero, 0)
        rem = zpt % ECHUNK
        if rem:
            pltpu.sync_copy(
                rows[0].at[pl.ds(0, rem)],
                acc.at[pl.ds(sid * zpt + (zpt // ECHUNK) * ECHUNK, rem)])
        plsc.subcore_barrier()

        def _gather(j, b):
            pltpu.async_copy(hp_hbm.at[src_v.at[j]], rows[b], gsem[b])

        def _gather_wait(j, b):
            pltpu.make_async_copy(hp_hbm.at[src_v.at[j]], rows[b],
                                  gsem[b]).wait()

        def _scatter(j, b):
            pltpu.async_copy(rows[b], acc.at[dst_v.at[j]], ssem[b], add=True)

        def _scatter_wait(j, b):
            pltpu.make_async_copy(rows[b], acc.at[dst_v.at[j]],
                                  ssem[b]).wait()

        for j0, cnt in phases:
            pltpu.sync_copy(src_hbm.at[wid, pl.ds(j0, cnt)],
                            src_v.at[pl.ds(0, cnt)])
            pltpu.sync_copy(dst_hbm.at[wid, pl.ds(j0, cnt)],
                            dst_v.at[pl.ds(0, cnt)])
            _gather(0, 0)
            _gather(1, 1)

            def _super(jj, carry):
                for b in range(3):
                    j = 3 * jj + b
                    fb = (b + 2) % 3

                    if b == 0:
                        @pl.when(jj > 0)
                        def _w():
                            _scatter_wait(j - 1, fb)
                    else:
                        _scatter_wait(j - 1, fb)

                    @pl.when(j + 2 < cnt)
                    def _g():
                        _gather(j + 2, fb)

                    _gather_wait(j, b)
                    _scatter(j, b)
                return carry

            lax.fori_loop(0, cnt // 3, _super, 0)
            _scatter_wait(cnt - 1, 2)
        plsc.subcore_barrier()

        pltpu.sync_copy(acc.at[pl.ds(sid * zpt, zpt)],
                        p_hbm.at[cid, pl.ds(sid * zpt, zpt)])

    return edge_kernel


def _mm_body(x_ref, w_ref, dg_ref, hp_ref):
    deg = dg_ref[0, :, 0:1] + dg_ref[1, :, 0:1] + 1.0
    dinv = lax.rsqrt(deg)
    h = jnp.dot(x_ref[...], w_ref[...], preferred_element_type=jnp.float32)
    hp_ref[...] = h * dinv


def _matmul_scaled(x, w, degp, rows_blk=1000):
    n, d_in = x.shape
    d_out = w.shape[1]
    grid = n // rows_blk
    return pl.pallas_call(
        _mm_body,
        grid=(grid,),
        in_specs=[
            pl.BlockSpec((rows_blk, d_in), lambda i: (i, 0)),
            pl.BlockSpec((d_in, d_out), lambda i: (0, 0)),
            pl.BlockSpec((2, rows_blk, 16), lambda i: (0, i, 0)),
        ],
        out_specs=pl.BlockSpec((rows_blk, d_out), lambda i: (i, 0)),
        out_shape=jax.ShapeDtypeStruct((n, d_out), jnp.float32),
    )(x, w, degp)


def _post_body(p_ref, hp_ref, dg_ref, b_ref, g_ref, bt_ref, out_ref):
    n = hp_ref.shape[0]
    deg = dg_ref[0, 0:n, 0:1] + dg_ref[1, 0:n, 0:1] + 1.0
    dinv = lax.rsqrt(deg)
    o = (p_ref[0, 0:n, :] + p_ref[1, 0:n, :] + hp_ref[...]) * dinv + b_ref[...]
    o = jnp.maximum(o, 0.0)
    m = jnp.sum(o, axis=0, keepdims=True) * (1.0 / n)
    c = o - m
    v = jnp.sum(c * c, axis=0, keepdims=True) * (1.0 / n)
    out_ref[...] = c * lax.rsqrt(v + BN_EPS) * g_ref[...] + bt_ref[...]


def _postprocess(p, hp, degp, bias, gamma, beta):
    n, d = hp.shape
    return pl.pallas_call(
        _post_body,
        out_shape=jax.ShapeDtypeStruct((n, d), jnp.float32),
    )(p, hp, degp, bias.reshape(1, d), gamma.reshape(1, d), beta.reshape(1, d))


def kernel(x, edge_index, W, bias, gamma, beta):
    n, d_in = x.shape
    d = W.shape[1]
    e = edge_index.shape[1]

    src = edge_index[0]
    dst = edge_index[1]

    kd = -(-e // (NW * CHUNK))
    ar_deg = -(-(n + 1) // (NS * CHUNK)) * (NS * CHUNK)
    pad_d = NW * kd * CHUNK - e
    dst_d = dst
    if pad_d:
        fill = jnp.arange(pad_d, dtype=jnp.int32)
        dst_d = jnp.concatenate([dst, n + fill % (ar_deg - n)])
    dst3 = dst_d.reshape(NW, kd, CHUNK)

    ke = -(-e // (NW * ECHUNK))
    ar_edge = -(-(n + 1) // (NS * 8)) * (NS * 8)
    pad_e = NW * ke * ECHUNK - e
    src_e, dst_e = src, dst
    if pad_e:
        fill = jnp.arange(pad_e, dtype=jnp.int32)
        src_e = jnp.concatenate([src, fill % n])
        dst_e = jnp.concatenate([dst, n + fill % (ar_edge - n)])
    src2 = src_e.reshape(NW, ke, ECHUNK)
    dst2 = dst_e.reshape(NW, ke, ECHUNK)

    degp = _make_deg_kernel(n, kd, ar_deg)(dst3)
    hp = _matmul_scaled(x, W, degp)
    p = _make_edge_kernel(n, d, ke, ar_edge)(hp, src2, dst2)
    return _postprocess(p, hp, degp, bias, gamma, beta)
